# baseline (speedup 1.0000x reference)
import functools

import jax
import jax.numpy as jnp
from jax import lax
from jax.experimental import pallas as pl
from jax.experimental.pallas import tpu as pltpu
from jax.experimental.pallas import tpu_sc as plsc

B = 16384
L = 50
D = 16
W_SHORT = 0.5

_NC = 2
_NS = 16
_NW = _NC * _NS

_SEQ_CH = 1600

_NEG_BIG = float(-2.0**32 + 1.0)


def _make_sc_body(n_ch, seq_per_w, small_per_w):
  def _sc_gather_body(clk_hbm, user_hbm, pos_hbm, neg_hbm,
                      user_t, item_t, item2_t,
                      seq_out, user_out, pos_out, neg_out, pos2_out, neg2_out,
                      idx_v, rows_v, sidx_v, srow_v, sem):
    c = lax.axis_index("c")
    s = lax.axis_index("s")
    wid = s * _NC + c

    def chunk(i, carry):
      base = wid * seq_per_w + i * _SEQ_CH
      sl = pl.ds(base, _SEQ_CH)
      pltpu.sync_copy(clk_hbm.at[sl], idx_v)
      pltpu.async_copy(item_t.at[idx_v], rows_v, sem).wait()
      pltpu.sync_copy(rows_v, seq_out.at[sl])
      return carry

    lax.fori_loop(0, n_ch, chunk, 0)

    sl = pl.ds(wid * small_per_w, small_per_w)
    pltpu.sync_copy(user_hbm.at[sl], sidx_v)
    pltpu.async_copy(user_t.at[sidx_v], srow_v, sem).wait()
    pltpu.sync_copy(srow_v, user_out.at[sl])

    pltpu.sync_copy(pos_hbm.at[sl], sidx_v)
    pltpu.async_copy(item_t.at[sidx_v], srow_v, sem).wait()
    pltpu.sync_copy(srow_v, pos_out.at[sl])
    pltpu.async_copy(item2_t.at[sidx_v], srow_v, sem).wait()
    pltpu.sync_copy(srow_v, pos2_out.at[sl])

    pltpu.sync_copy(neg_hbm.at[sl], sidx_v)
    pltpu.async_copy(item_t.at[sidx_v], srow_v, sem).wait()
    pltpu.sync_copy(srow_v, neg_out.at[sl])
    pltpu.async_copy(item2_t.at[sidx_v], srow_v, sem).wait()
    pltpu.sync_copy(srow_v, neg2_out.at[sl])

  return _sc_gather_body


def _sc_gather(clk_flat, user, pos, neg_flat, user_table, item_table, item2_table):
  nb = user.shape[0]
  seq_per_w = nb * L // _NW
  n_ch = seq_per_w // _SEQ_CH
  small_per_w = nb // _NW
  row = jax.ShapeDtypeStruct((nb, D), jnp.float32)
  out_type = (jax.ShapeDtypeStruct((nb * L, D), jnp.float32),
              row, row, row, row, row)
  mesh = plsc.VectorSubcoreMesh(core_axis_name="c", subcore_axis_name="s")
  f = pl.kernel(
      _make_sc_body(n_ch, seq_per_w, small_per_w),
      out_type=out_type,
      mesh=mesh,
      scratch_types=[
          pltpu.VMEM((_SEQ_CH,), jnp.int32),
          pltpu.VMEM((_SEQ_CH, D), jnp.float32),
          pltpu.VMEM((small_per_w,), jnp.int32),
          pltpu.VMEM((small_per_w, D), jnp.float32),
          pltpu.SemaphoreType.DMA,
      ],
      compiler_params=pltpu.CompilerParams(use_tc_tiling_on_sc=False),
  )
  return f(clk_flat, user, pos, neg_flat, user_table, item_table, item2_table)


_BB = 128


def _att_body(seq_ref, clk_ref, u_ref, pe_ref, ne_ref, p2_ref, n2_ref, w_ref,
              out_ref):
  x = seq_ref[...]
  w = w_ref[...] * 0.5
  wb = lax.broadcast_in_dim(w, (_BB, D, D), (1, 2))
  k = jnp.maximum(
      lax.dot_general(x, wb, (((2,), (1,)), ((0,), (0,))),
                      preferred_element_type=jnp.float32), 0.0)
  s = lax.dot_general(k, k, (((2,), (2,)), ((0,), (0,))),
                      preferred_element_type=jnp.float32)
  qmask = clk_ref[...] != 0
  e = jnp.where(qmask, jnp.exp(jnp.minimum(s, 60.0)), 1.0)
  z = jnp.sum(e, axis=-1, keepdims=True)
  y = lax.dot_general(e, x, (((2,), (1,)), ((0,), (0,))),
                      preferred_element_type=jnp.float32)
  si = jnp.sum(y * (1.0 / (z * float(L))), axis=1)

  u = u_ref[...]
  pos = (W_SHORT * jnp.sum(u * p2_ref[...], axis=-1, keepdims=True)
         + (1.0 - W_SHORT) * jnp.sum(si * pe_ref[...], axis=-1, keepdims=True))
  neg = (W_SHORT * jnp.sum(u * n2_ref[...], axis=-1, keepdims=True)
         + (1.0 - W_SHORT) * jnp.sum(si * ne_ref[...], axis=-1, keepdims=True))
  out_ref[...] = jnp.concatenate([pos, neg], axis=-1)


def _att_call(seq3, click_seq, user_e, pos_e, neg_e, pos2_e, neg2_e, w_att):
  nb = seq3.shape[0]
  grid = nb // _BB
  row_spec = pl.BlockSpec((_BB, D), lambda i: (i, 0))
  return pl.pallas_call(
      _att_body,
      grid=(grid,),
      in_specs=[
          pl.BlockSpec((_BB, L, D), lambda i: (i, 0, 0)),
          pl.BlockSpec((_BB, L, 1), lambda i: (i, 0, 0)),
          row_spec, row_spec, row_spec, row_spec, row_spec,
          pl.BlockSpec((D, D), lambda i: (0, 0)),
      ],
      out_specs=pl.BlockSpec((_BB, 2), lambda i: (i, 0)),
      out_shape=jax.ShapeDtypeStruct((nb, 2), jnp.float32),
  )(seq3, click_seq.reshape(nb, L, 1), user_e, pos_e, neg_e, pos2_e, neg2_e,
    w_att)


_N_SPLIT = 1


def kernel(user, click_seq, pos_item, neg_item, user_table, item_table,
           item2_table, W_att):
  h = B // _N_SPLIT
  outs = []
  gathered = []
  for i in range(_N_SPLIT):
    sl = slice(i * h, (i + 1) * h)
    gathered.append(
        (_sc_gather(click_seq[sl].reshape(-1), user[sl], pos_item[sl],
                    neg_item[sl].reshape(-1), user_table, item_table,
                    item2_table), click_seq[sl]))
  for (seq_e, user_e, pos_e, neg_e, pos2_e, neg2_e), clk in gathered:
    outs.append(_att_call(seq_e.reshape(h, L, D), clk, user_e, pos_e, neg_e,
                          pos2_e, neg2_e, W_att))
  return jnp.concatenate(outs, axis=0)


# speedup vs baseline: 1.0078x; 1.0078x over previous
import functools

import jax
import jax.numpy as jnp
from jax import lax
from jax.experimental import pallas as pl
from jax.experimental.pallas import tpu as pltpu
from jax.experimental.pallas import tpu_sc as plsc

B = 16384
L = 50
D = 16
W_SHORT = 0.5

_NC = 2
_NS = 16
_NW = _NC * _NS

_SEQ_CH = 1600

_NEG_BIG = float(-2.0**32 + 1.0)


def _make_sc_body(n_ch, seq_per_w, small_per_w):
  def _sc_gather_body(clk_hbm, user_hbm, pos_hbm, neg_hbm,
                      user_t, item_t, item2_t,
                      seq_out, user_out, pos_out, neg_out, pos2_out, neg2_out,
                      idx_v, rows_v, sidx_v, srow_v, sem):
    c = lax.axis_index("c")
    s = lax.axis_index("s")
    wid = s * _NC + c

    def chunk(i, carry):
      base = wid * seq_per_w + i * _SEQ_CH
      sl = pl.ds(base, _SEQ_CH)
      pltpu.sync_copy(clk_hbm.at[sl], idx_v)
      pltpu.async_copy(item_t.at[idx_v], rows_v, sem).wait()
      pltpu.sync_copy(rows_v, seq_out.at[sl])
      return carry

    lax.fori_loop(0, n_ch, chunk, 0)

    sl = pl.ds(wid * small_per_w, small_per_w)
    pltpu.sync_copy(user_hbm.at[sl], sidx_v)
    pltpu.async_copy(user_t.at[sidx_v], srow_v, sem).wait()
    pltpu.sync_copy(srow_v, user_out.at[sl])

    pltpu.sync_copy(pos_hbm.at[sl], sidx_v)
    pltpu.async_copy(item_t.at[sidx_v], srow_v, sem).wait()
    pltpu.sync_copy(srow_v, pos_out.at[sl])
    pltpu.async_copy(item2_t.at[sidx_v], srow_v, sem).wait()
    pltpu.sync_copy(srow_v, pos2_out.at[sl])

    pltpu.sync_copy(neg_hbm.at[sl], sidx_v)
    pltpu.async_copy(item_t.at[sidx_v], srow_v, sem).wait()
    pltpu.sync_copy(srow_v, neg_out.at[sl])
    pltpu.async_copy(item2_t.at[sidx_v], srow_v, sem).wait()
    pltpu.sync_copy(srow_v, neg2_out.at[sl])

  return _sc_gather_body


def _sc_gather(clk_flat, user, pos, neg_flat, user_table, item_table, item2_table):
  nb = user.shape[0]
  seq_per_w = nb * L // _NW
  n_ch = seq_per_w // _SEQ_CH
  small_per_w = nb // _NW
  row = jax.ShapeDtypeStruct((nb, D), jnp.float32)
  out_type = (jax.ShapeDtypeStruct((nb * L, D), jnp.float32),
              row, row, row, row, row)
  mesh = plsc.VectorSubcoreMesh(core_axis_name="c", subcore_axis_name="s")
  f = pl.kernel(
      _make_sc_body(n_ch, seq_per_w, small_per_w),
      out_type=out_type,
      mesh=mesh,
      scratch_types=[
          pltpu.VMEM((_SEQ_CH,), jnp.int32),
          pltpu.VMEM((_SEQ_CH, D), jnp.float32),
          pltpu.VMEM((small_per_w,), jnp.int32),
          pltpu.VMEM((small_per_w, D), jnp.float32),
          pltpu.SemaphoreType.DMA,
      ],
      compiler_params=pltpu.CompilerParams(use_tc_tiling_on_sc=False),
  )
  return f(clk_flat, user, pos, neg_flat, user_table, item_table, item2_table)


_BB = 256


def _att_body(seq_ref, clk_ref, u_ref, pe_ref, ne_ref, p2_ref, n2_ref, w_ref,
              out_ref):
  x = seq_ref[...]
  w = w_ref[...] * 0.5
  wb = lax.broadcast_in_dim(w, (_BB, D, D), (1, 2))
  k = jnp.maximum(
      lax.dot_general(x, wb, (((2,), (1,)), ((0,), (0,))),
                      preferred_element_type=jnp.float32), 0.0)
  s = lax.dot_general(k, k, (((2,), (2,)), ((0,), (0,))),
                      preferred_element_type=jnp.float32)
  qmask = clk_ref[...] != 0
  e = jnp.where(qmask, jnp.exp(jnp.minimum(s, 60.0)), 1.0)
  z = jnp.sum(e, axis=-1, keepdims=True)
  y = lax.dot_general(e, x, (((2,), (1,)), ((0,), (0,))),
                      preferred_element_type=jnp.float32)
  si = jnp.sum(y * (1.0 / (z * float(L))), axis=1)

  u = u_ref[...]
  pos = (W_SHORT * jnp.sum(u * p2_ref[...], axis=-1, keepdims=True)
         + (1.0 - W_SHORT) * jnp.sum(si * pe_ref[...], axis=-1, keepdims=True))
  neg = (W_SHORT * jnp.sum(u * n2_ref[...], axis=-1, keepdims=True)
         + (1.0 - W_SHORT) * jnp.sum(si * ne_ref[...], axis=-1, keepdims=True))
  out_ref[...] = jnp.concatenate([pos, neg], axis=-1)


def _att_call(seq3, click_seq, user_e, pos_e, neg_e, pos2_e, neg2_e, w_att):
  nb = seq3.shape[0]
  grid = nb // _BB
  row_spec = pl.BlockSpec((_BB, D), lambda i: (i, 0))
  return pl.pallas_call(
      _att_body,
      grid=(grid,),
      in_specs=[
          pl.BlockSpec((_BB, L, D), lambda i: (i, 0, 0)),
          pl.BlockSpec((_BB, L, 1), lambda i: (i, 0, 0)),
          row_spec, row_spec, row_spec, row_spec, row_spec,
          pl.BlockSpec((D, D), lambda i: (0, 0)),
      ],
      out_specs=pl.BlockSpec((_BB, 2), lambda i: (i, 0)),
      out_shape=jax.ShapeDtypeStruct((nb, 2), jnp.float32),
  )(seq3, click_seq.reshape(nb, L, 1), user_e, pos_e, neg_e, pos2_e, neg2_e,
    w_att)


_N_SPLIT = 1


def kernel(user, click_seq, pos_item, neg_item, user_table, item_table,
           item2_table, W_att):
  h = B // _N_SPLIT
  outs = []
  gathered = []
  for i in range(_N_SPLIT):
    sl = slice(i * h, (i + 1) * h)
    gathered.append(
        (_sc_gather(click_seq[sl].reshape(-1), user[sl], pos_item[sl],
                    neg_item[sl].reshape(-1), user_table, item_table,
                    item2_table), click_seq[sl]))
  for (seq_e, user_e, pos_e, neg_e, pos2_e, neg2_e), clk in gathered:
    outs.append(_att_call(seq_e.reshape(h, L, D), clk, user_e, pos_e, neg_e,
                          pos2_e, neg2_e, W_att))
  return jnp.concatenate(outs, axis=0)
